# baseline (speedup 1.0000x reference)
import functools

import jax
import jax.numpy as jnp
from jax import lax
from jax.experimental import pallas as pl
from jax.experimental.pallas import tpu as pltpu
from jax.experimental.pallas import tpu_sc as plsc

N = 10000
E = 320000
D = 128
HID = 256
G = 64

NC = 2
NS = 16
NW = NC * NS

CH = 128
CPT = 80
TOT = CPT * NW
GS = 16
EP = CH * CPT
E_PAD = EP * NW
N_PAD = 10240
RPT = 624
RREM = N - NS * RPT
RBASE = NS * RPT
DRT = N_PAD // NS

_MESH = plsc.VectorSubcoreMesh(
    core_axis_name="c", subcore_axis_name="s", num_cores=NC, num_subcores=NS)


DW = 128


@functools.partial(
    pl.kernel,
    out_type=jax.ShapeDtypeStruct((NC, N_PAD, DW), jnp.float32),
    mesh=_MESH,
    scratch_types=[
        pltpu.VMEM((CPT, CH), jnp.int32),
        pltpu.VMEM((CH, DW), jnp.float32),
        pltpu.VMEM_SHARED((N_PAD, DW), jnp.float32),
    ],
)
def _sc_deg(dstp_hbm, zeros_hbm, ones_hbm, out_hbm, dst_v, ones_v, acc_sh):
    c = lax.axis_index("c")
    s = lax.axis_index("s")
    wid = s * NC + c
    d0 = pl.multiple_of(s * DRT, 8)
    pltpu.sync_copy(zeros_hbm.at[pl.ds(d0, DRT)],
                    acc_sh.at[pl.ds(d0, DRT)])
    pltpu.sync_copy(ones_hbm, ones_v)
    pltpu.sync_copy(dstp_hbm.at[wid], dst_v)
    plsc.subcore_barrier()

    def body(j, carry):
        pltpu.sync_copy(ones_v, acc_sh.at[dst_v.at[j]], add=True)
        return carry

    lax.fori_loop(0, CPT, body, 0)
    plsc.subcore_barrier()
    pltpu.sync_copy(acc_sh.at[pl.ds(d0, DRT)],
                    out_hbm.at[c, pl.ds(d0, DRT)])


@functools.partial(
    pl.kernel,
    out_type=jax.ShapeDtypeStruct((NC, N, D), jnp.float32),
    mesh=_MESH,
    scratch_types=[
        pltpu.VMEM((GS, CH), jnp.int32),
        pltpu.VMEM((GS, CH), jnp.int32),
        pltpu.VMEM((CH, D), jnp.float32),
        pltpu.VMEM((CH, D), jnp.float32),
        pltpu.VMEM_SHARED((N_PAD, D), jnp.float32),
        pltpu.SemaphoreType.DMA,
        pltpu.SemaphoreType.DMA,
    ],
)
def _sc_agg(srcp_hbm, dstp_hbm, y_hbm, out_hbm, src_v, dst_v, buf0, buf1,
            acc_sh, sem0, sem1):
    c = lax.axis_index("c")
    s = lax.axis_index("s")
    r0 = pl.multiple_of(s * RPT, 8)
    pltpu.sync_copy(y_hbm.at[pl.ds(r0, RPT)], acc_sh.at[pl.ds(r0, RPT)])

    @pl.when(s == 0)
    def _seed_rem():
        pltpu.sync_copy(y_hbm.at[pl.ds(RBASE, RREM)],
                        acc_sh.at[pl.ds(RBASE, RREM)])
        pltpu.sync_copy(y_hbm.at[pl.ds(0, N_PAD - N)],
                        acc_sh.at[pl.ds(N, N_PAD - N)])

    plsc.subcore_barrier()

    def run_edges(base, nstage):
        for g in range(nstage):
            off = pl.multiple_of(base + g * GS, 8)
            pltpu.sync_copy(srcp_hbm.at[pl.ds(off, GS)], src_v)
            pltpu.sync_copy(dstp_hbm.at[pl.ds(off, GS)], dst_v)
            pltpu.async_copy(y_hbm.at[src_v.at[0]], buf0, sem0)

            def body(k, carry):
                j0 = 2 * k
                pltpu.async_copy(y_hbm.at[src_v.at[j0 + 1]], buf1, sem1)
                pltpu.make_async_copy(y_hbm.at[src_v.at[j0]], buf0, sem0).wait()
                pltpu.sync_copy(buf0, acc_sh.at[dst_v.at[j0]], add=True)
                pltpu.async_copy(y_hbm.at[src_v.at[(j0 + 2) % GS]], buf0, sem0)
                pltpu.make_async_copy(y_hbm.at[src_v.at[j0 + 1]], buf1,
                                      sem1).wait()
                pltpu.sync_copy(buf1, acc_sh.at[dst_v.at[j0 + 1]], add=True)
                return carry

            lax.fori_loop(0, GS // 2, body, 0)
            pltpu.make_async_copy(y_hbm.at[src_v.at[0]], buf0, sem0).wait()

    run_edges((s * NC + c) * CPT, CPT // GS)
    plsc.subcore_barrier()
    pltpu.sync_copy(acc_sh.at[pl.ds(r0, RPT)],
                    out_hbm.at[c, pl.ds(r0, RPT)])

    @pl.when(s == 0)
    def _out_rem():
        pltpu.sync_copy(acc_sh.at[pl.ds(RBASE, RREM)],
                        out_hbm.at[c, pl.ds(RBASE, RREM)])


_BN = 1000
_GRID = N // _BN


def _tc_first_body(degp_ref, x_ref, w_ref, dis_ref, y_ref):
    deg = degp_ref[0] + degp_ref[1] + 1.0
    dis = lax.rsqrt(deg)
    dis_ref[...] = dis
    y_ref[...] = dis * jnp.dot(x_ref[...], w_ref[...],
                               preferred_element_type=jnp.float32)


def _tc_first(degp, x, w):
    return pl.pallas_call(
        _tc_first_body,
        grid=(_GRID,),
        in_specs=[
            pl.BlockSpec((NC, _BN, 1), lambda i: (0, i, 0)),
            pl.BlockSpec((_BN, D), lambda i: (i, 0)),
            pl.BlockSpec((D, D), lambda i: (0, 0)),
        ],
        out_specs=[
            pl.BlockSpec((_BN, 1), lambda i: (i, 0)),
            pl.BlockSpec((_BN, D), lambda i: (i, 0)),
        ],
        out_shape=[
            jax.ShapeDtypeStruct((N, 1), jnp.float32),
            jax.ShapeDtypeStruct((N, D), jnp.float32),
        ],
    )(degp, x, w)


def _tc_mid_body(z_ref, yin_ref, dis_ref, b_ref, w_ref, y_ref):
    h = jnp.maximum(
        dis_ref[...] * (z_ref[0] + z_ref[1] - yin_ref[...]) + b_ref[...], 0.0)
    y_ref[...] = dis_ref[...] * jnp.dot(h, w_ref[...],
                                        preferred_element_type=jnp.float32)


def _tc_mid(z, yin, dis, b2d, w):
    return pl.pallas_call(
        _tc_mid_body,
        grid=(_GRID,),
        in_specs=[
            pl.BlockSpec((NC, _BN, D), lambda i: (0, i, 0)),
            pl.BlockSpec((_BN, D), lambda i: (i, 0)),
            pl.BlockSpec((_BN, 1), lambda i: (i, 0)),
            pl.BlockSpec((1, D), lambda i: (0, 0)),
            pl.BlockSpec((D, D), lambda i: (0, 0)),
        ],
        out_specs=pl.BlockSpec((_BN, D), lambda i: (i, 0)),
        out_shape=jax.ShapeDtypeStruct((N, D), jnp.float32),
    )(z, yin, dis, b2d, w)


def _tc_final_body(z_ref, yin_ref, dis_ref, b_ref, batch_ref, l1w_ref,
                   l1b_ref, l2w_ref, l2b_ref, out_ref, sums, counts):
    i = pl.program_id(0)

    @pl.when(i == 0)
    def _init():
        sums[...] = jnp.zeros_like(sums)
        counts[...] = jnp.zeros_like(counts)

    h = jnp.maximum(
        dis_ref[...] * (z_ref[0] + z_ref[1] - yin_ref[...]) + b_ref[...], 0.0)
    iota = lax.broadcasted_iota(jnp.int32, (G, 1), 0)
    onehot = (batch_ref[0] == iota).astype(jnp.float32)
    sums[...] += jnp.dot(onehot, h, preferred_element_type=jnp.float32)
    counts[...] += jnp.dot(onehot, jnp.ones((_BN, 1), jnp.float32),
                           preferred_element_type=jnp.float32)

    @pl.when(i == _GRID - 1)
    def _finish():
        pooled = sums[...] / jnp.maximum(counts[...], 1.0)
        a = jnp.maximum(
            jnp.dot(pooled, l1w_ref[...], preferred_element_type=jnp.float32)
            + l1b_ref[...], 0.0)
        out_ref[...] = (jnp.dot(a, l2w_ref[...],
                                preferred_element_type=jnp.float32)
                        + l2b_ref[...])


def _tc_final(z, yin, dis, b2d, batch3, l1w, l1b2d, l2w, l2b2d):
    return pl.pallas_call(
        _tc_final_body,
        grid=(_GRID,),
        in_specs=[
            pl.BlockSpec((NC, _BN, D), lambda i: (0, i, 0)),
            pl.BlockSpec((_BN, D), lambda i: (i, 0)),
            pl.BlockSpec((_BN, 1), lambda i: (i, 0)),
            pl.BlockSpec((1, D), lambda i: (0, 0)),
            pl.BlockSpec((1, 1, _BN), lambda i: (i, 0, 0)),
            pl.BlockSpec((D, HID), lambda i: (0, 0)),
            pl.BlockSpec((1, HID), lambda i: (0, 0)),
            pl.BlockSpec((HID, 1), lambda i: (0, 0)),
            pl.BlockSpec((1, 1), lambda i: (0, 0)),
        ],
        out_specs=pl.BlockSpec((G, 1), lambda i: (0, 0)),
        out_shape=jax.ShapeDtypeStruct((G, 1), jnp.float32),
        scratch_shapes=[
            pltpu.VMEM((G, D), jnp.float32),
            pltpu.VMEM((G, 1), jnp.float32),
        ],
    )(z, yin, dis, b2d, batch3, l1w, l1b2d, l2w, l2b2d)


def kernel(x, edge_index, batch, W1, b1, W2, b2, W3, b3,
           lin1_W, lin1_b, lin2_W, lin2_b):
    src = edge_index[0]
    dst = edge_index[1]
    pad = E_PAD - E
    srcp = jnp.concatenate(
        [src, jnp.zeros((pad,), jnp.int32)]).reshape(TOT, CH)
    pad_dst = N + (jnp.arange(pad, dtype=jnp.int32) % (N_PAD - N))
    dstp = jnp.concatenate([dst, pad_dst]).reshape(TOT, CH)
    dstp3 = dstp.reshape(NW, CPT, CH)
    zeros_deg = jnp.zeros((N_PAD, DW), jnp.float32)
    ones_ch = jnp.ones((CH, DW), jnp.float32)
    batch3 = batch.reshape(_GRID, 1, _BN)

    degp = _sc_deg(dstp3, zeros_deg, ones_ch)
    degp = degp[:, :, :1]
    dis, y = _tc_first(degp, x, W1)
    z = _sc_agg(srcp, dstp, y)
    y = _tc_mid(z, y, dis, b1.reshape(1, D), W2)
    z = _sc_agg(srcp, dstp, y)
    y = _tc_mid(z, y, dis, b2.reshape(1, D), W3)
    z = _sc_agg(srcp, dstp, y)
    return _tc_final(z, y, dis, b3.reshape(1, D), batch3,
                     lin1_W, lin1_b.reshape(1, HID),
                     lin2_W, lin2_b.reshape(1, 1))


# speedup vs baseline: 1.0394x; 1.0394x over previous
import functools

import jax
import jax.numpy as jnp
from jax import lax
from jax.experimental import pallas as pl
from jax.experimental.pallas import tpu as pltpu
from jax.experimental.pallas import tpu_sc as plsc

N = 10000
E = 320000
D = 128
HID = 256
G = 64

NC = 2
NS = 16
NW = NC * NS

CH = 128
CPT = 80
TOT = CPT * NW
GS = 40
EP = CH * CPT
E_PAD = EP * NW
N_PAD = 10240
RPT = 624
RREM = N - NS * RPT
RBASE = NS * RPT
DRT = N_PAD // NS

_MESH = plsc.VectorSubcoreMesh(
    core_axis_name="c", subcore_axis_name="s", num_cores=NC, num_subcores=NS)


DW = 128


@functools.partial(
    pl.kernel,
    out_type=jax.ShapeDtypeStruct((NC, N_PAD, DW), jnp.float32),
    mesh=_MESH,
    scratch_types=[
        pltpu.VMEM((CPT, CH), jnp.int32),
        pltpu.VMEM((CH, DW), jnp.float32),
        pltpu.VMEM_SHARED((N_PAD, DW), jnp.float32),
    ],
)
def _sc_deg(dstp_hbm, zeros_hbm, ones_hbm, out_hbm, dst_v, ones_v, acc_sh):
    c = lax.axis_index("c")
    s = lax.axis_index("s")
    wid = s * NC + c
    d0 = pl.multiple_of(s * DRT, 8)
    pltpu.sync_copy(zeros_hbm.at[pl.ds(d0, DRT)],
                    acc_sh.at[pl.ds(d0, DRT)])
    pltpu.sync_copy(ones_hbm, ones_v)
    pltpu.sync_copy(dstp_hbm.at[wid], dst_v)
    plsc.subcore_barrier()

    def body(j, carry):
        pltpu.sync_copy(ones_v, acc_sh.at[dst_v.at[j]], add=True)
        return carry

    lax.fori_loop(0, CPT, body, 0)
    plsc.subcore_barrier()
    pltpu.sync_copy(acc_sh.at[pl.ds(d0, DRT)],
                    out_hbm.at[c, pl.ds(d0, DRT)])


@functools.partial(
    pl.kernel,
    out_type=jax.ShapeDtypeStruct((NC, N, D), jnp.float32),
    mesh=_MESH,
    scratch_types=[
        pltpu.VMEM((GS, CH), jnp.int32),
        pltpu.VMEM((GS, CH), jnp.int32),
        pltpu.VMEM((CH, D), jnp.float32),
        pltpu.VMEM((CH, D), jnp.float32),
        pltpu.VMEM_SHARED((N_PAD, D), jnp.float32),
        pltpu.SemaphoreType.DMA,
        pltpu.SemaphoreType.DMA,
    ],
)
def _sc_agg(srcp_hbm, dstp_hbm, y_hbm, out_hbm, src_v, dst_v, buf0, buf1,
            acc_sh, sem0, sem1):
    c = lax.axis_index("c")
    s = lax.axis_index("s")
    r0 = pl.multiple_of(s * RPT, 8)
    pltpu.sync_copy(y_hbm.at[pl.ds(r0, RPT)], acc_sh.at[pl.ds(r0, RPT)])

    @pl.when(s == 0)
    def _seed_rem():
        pltpu.sync_copy(y_hbm.at[pl.ds(RBASE, RREM)],
                        acc_sh.at[pl.ds(RBASE, RREM)])
        pltpu.sync_copy(y_hbm.at[pl.ds(0, N_PAD - N)],
                        acc_sh.at[pl.ds(N, N_PAD - N)])

    plsc.subcore_barrier()

    def run_edges(base, nstage):
        for g in range(nstage):
            off = pl.multiple_of(base + g * GS, 8)
            pltpu.sync_copy(srcp_hbm.at[pl.ds(off, GS)], src_v)
            pltpu.sync_copy(dstp_hbm.at[pl.ds(off, GS)], dst_v)
            pltpu.async_copy(y_hbm.at[src_v.at[0]], buf0, sem0)

            def body(k, carry):
                j0 = 2 * k
                pltpu.async_copy(y_hbm.at[src_v.at[j0 + 1]], buf1, sem1)
                pltpu.make_async_copy(y_hbm.at[src_v.at[j0]], buf0, sem0).wait()
                pltpu.sync_copy(buf0, acc_sh.at[dst_v.at[j0]], add=True)
                pltpu.async_copy(y_hbm.at[src_v.at[(j0 + 2) % GS]], buf0, sem0)
                pltpu.make_async_copy(y_hbm.at[src_v.at[j0 + 1]], buf1,
                                      sem1).wait()
                pltpu.sync_copy(buf1, acc_sh.at[dst_v.at[j0 + 1]], add=True)
                return carry

            lax.fori_loop(0, GS // 2, body, 0)
            pltpu.make_async_copy(y_hbm.at[src_v.at[0]], buf0, sem0).wait()

    run_edges((s * NC + c) * CPT, CPT // GS)
    plsc.subcore_barrier()
    pltpu.sync_copy(acc_sh.at[pl.ds(r0, RPT)],
                    out_hbm.at[c, pl.ds(r0, RPT)])

    @pl.when(s == 0)
    def _out_rem():
        pltpu.sync_copy(acc_sh.at[pl.ds(RBASE, RREM)],
                        out_hbm.at[c, pl.ds(RBASE, RREM)])


_BN = 1000
_GRID = N // _BN


def _tc_first_body(degp_ref, x_ref, w_ref, dis_ref, y_ref):
    deg = degp_ref[0] + degp_ref[1] + 1.0
    dis = lax.rsqrt(deg)
    dis_ref[...] = dis
    y_ref[...] = dis * jnp.dot(x_ref[...], w_ref[...],
                               preferred_element_type=jnp.float32)


def _tc_first(degp, x, w):
    return pl.pallas_call(
        _tc_first_body,
        grid=(_GRID,),
        in_specs=[
            pl.BlockSpec((NC, _BN, 1), lambda i: (0, i, 0)),
            pl.BlockSpec((_BN, D), lambda i: (i, 0)),
            pl.BlockSpec((D, D), lambda i: (0, 0)),
        ],
        out_specs=[
            pl.BlockSpec((_BN, 1), lambda i: (i, 0)),
            pl.BlockSpec((_BN, D), lambda i: (i, 0)),
        ],
        out_shape=[
            jax.ShapeDtypeStruct((N, 1), jnp.float32),
            jax.ShapeDtypeStruct((N, D), jnp.float32),
        ],
    )(degp, x, w)


def _tc_mid_body(z_ref, yin_ref, dis_ref, b_ref, w_ref, y_ref):
    h = jnp.maximum(
        dis_ref[...] * (z_ref[0] + z_ref[1] - yin_ref[...]) + b_ref[...], 0.0)
    y_ref[...] = dis_ref[...] * jnp.dot(h, w_ref[...],
                                        preferred_element_type=jnp.float32)


def _tc_mid(z, yin, dis, b2d, w):
    return pl.pallas_call(
        _tc_mid_body,
        grid=(_GRID,),
        in_specs=[
            pl.BlockSpec((NC, _BN, D), lambda i: (0, i, 0)),
            pl.BlockSpec((_BN, D), lambda i: (i, 0)),
            pl.BlockSpec((_BN, 1), lambda i: (i, 0)),
            pl.BlockSpec((1, D), lambda i: (0, 0)),
            pl.BlockSpec((D, D), lambda i: (0, 0)),
        ],
        out_specs=pl.BlockSpec((_BN, D), lambda i: (i, 0)),
        out_shape=jax.ShapeDtypeStruct((N, D), jnp.float32),
    )(z, yin, dis, b2d, w)


def _tc_final_body(z_ref, yin_ref, dis_ref, b_ref, batch_ref, l1w_ref,
                   l1b_ref, l2w_ref, l2b_ref, out_ref, sums, counts):
    i = pl.program_id(0)

    @pl.when(i == 0)
    def _init():
        sums[...] = jnp.zeros_like(sums)
        counts[...] = jnp.zeros_like(counts)

    h = jnp.maximum(
        dis_ref[...] * (z_ref[0] + z_ref[1] - yin_ref[...]) + b_ref[...], 0.0)
    iota = lax.broadcasted_iota(jnp.int32, (G, 1), 0)
    onehot = (batch_ref[0] == iota).astype(jnp.float32)
    sums[...] += jnp.dot(onehot, h, preferred_element_type=jnp.float32)
    counts[...] += jnp.dot(onehot, jnp.ones((_BN, 1), jnp.float32),
                           preferred_element_type=jnp.float32)

    @pl.when(i == _GRID - 1)
    def _finish():
        pooled = sums[...] / jnp.maximum(counts[...], 1.0)
        a = jnp.maximum(
            jnp.dot(pooled, l1w_ref[...], preferred_element_type=jnp.float32)
            + l1b_ref[...], 0.0)
        out_ref[...] = (jnp.dot(a, l2w_ref[...],
                                preferred_element_type=jnp.float32)
                        + l2b_ref[...])


def _tc_final(z, yin, dis, b2d, batch3, l1w, l1b2d, l2w, l2b2d):
    return pl.pallas_call(
        _tc_final_body,
        grid=(_GRID,),
        in_specs=[
            pl.BlockSpec((NC, _BN, D), lambda i: (0, i, 0)),
            pl.BlockSpec((_BN, D), lambda i: (i, 0)),
            pl.BlockSpec((_BN, 1), lambda i: (i, 0)),
            pl.BlockSpec((1, D), lambda i: (0, 0)),
            pl.BlockSpec((1, 1, _BN), lambda i: (i, 0, 0)),
            pl.BlockSpec((D, HID), lambda i: (0, 0)),
            pl.BlockSpec((1, HID), lambda i: (0, 0)),
            pl.BlockSpec((HID, 1), lambda i: (0, 0)),
            pl.BlockSpec((1, 1), lambda i: (0, 0)),
        ],
        out_specs=pl.BlockSpec((G, 1), lambda i: (0, 0)),
        out_shape=jax.ShapeDtypeStruct((G, 1), jnp.float32),
        scratch_shapes=[
            pltpu.VMEM((G, D), jnp.float32),
            pltpu.VMEM((G, 1), jnp.float32),
        ],
    )(z, yin, dis, b2d, batch3, l1w, l1b2d, l2w, l2b2d)


def kernel(x, edge_index, batch, W1, b1, W2, b2, W3, b3,
           lin1_W, lin1_b, lin2_W, lin2_b):
    src = edge_index[0]
    dst = edge_index[1]
    pad = E_PAD - E
    srcp = jnp.concatenate(
        [src, jnp.zeros((pad,), jnp.int32)]).reshape(TOT, CH)
    pad_dst = N + (jnp.arange(pad, dtype=jnp.int32) % (N_PAD - N))
    dstp = jnp.concatenate([dst, pad_dst]).reshape(TOT, CH)
    dstp3 = dstp.reshape(NW, CPT, CH)
    zeros_deg = jnp.zeros((N_PAD, DW), jnp.float32)
    ones_ch = jnp.ones((CH, DW), jnp.float32)
    batch3 = batch.reshape(_GRID, 1, _BN)

    degp = _sc_deg(dstp3, zeros_deg, ones_ch)
    degp = degp[:, :, :1]
    dis, y = _tc_first(degp, x, W1)
    z = _sc_agg(srcp, dstp, y)
    y = _tc_mid(z, y, dis, b1.reshape(1, D), W2)
    z = _sc_agg(srcp, dstp, y)
    y = _tc_mid(z, y, dis, b2.reshape(1, D), W3)
    z = _sc_agg(srcp, dstp, y)
    return _tc_final(z, y, dis, b3.reshape(1, D), batch3,
                     lin1_W, lin1_b.reshape(1, HID),
                     lin2_W, lin2_b.reshape(1, 1))


# speedup vs baseline: 1.1244x; 1.0818x over previous
import functools

import jax
import jax.numpy as jnp
from jax import lax
from jax.experimental import pallas as pl
from jax.experimental.pallas import tpu as pltpu
from jax.experimental.pallas import tpu_sc as plsc

N = 10000
E = 320000
D = 128
HID = 256
G = 64

NC = 2
NS = 16
NW = NC * NS

CH = 128
CPT = 80
TOT = CPT * NW
GS = 40
EP = CH * CPT
E_PAD = EP * NW
N_PAD = 10240
RPT = 624
RREM = N - NS * RPT
RBASE = NS * RPT
DRT = N_PAD // NS

_MESH = plsc.VectorSubcoreMesh(
    core_axis_name="c", subcore_axis_name="s", num_cores=NC, num_subcores=NS)


DW = 128


@functools.partial(
    pl.kernel,
    out_type=jax.ShapeDtypeStruct((NC, N_PAD, DW), jnp.float32),
    mesh=_MESH,
    scratch_types=[
        pltpu.VMEM((CPT, CH), jnp.int32),
        pltpu.VMEM((CH, DW), jnp.float32),
        pltpu.VMEM_SHARED((N_PAD, DW), jnp.float32),
    ],
)
def _sc_deg(dstp_hbm, zeros_hbm, ones_hbm, out_hbm, dst_v, ones_v, acc_sh):
    c = lax.axis_index("c")
    s = lax.axis_index("s")
    wid = s * NC + c
    d0 = pl.multiple_of(s * DRT, 8)
    pltpu.sync_copy(zeros_hbm.at[pl.ds(d0, DRT)],
                    acc_sh.at[pl.ds(d0, DRT)])
    pltpu.sync_copy(ones_hbm, ones_v)
    pltpu.sync_copy(dstp_hbm.at[wid], dst_v)
    plsc.subcore_barrier()

    def body(j, carry):
        pltpu.sync_copy(ones_v, acc_sh.at[dst_v.at[j]], add=True)
        return carry

    lax.fori_loop(0, CPT, body, 0)
    plsc.subcore_barrier()
    pltpu.sync_copy(acc_sh.at[pl.ds(d0, DRT)],
                    out_hbm.at[c, pl.ds(d0, DRT)])


@functools.partial(
    pl.kernel,
    out_type=jax.ShapeDtypeStruct((NC, N, D), jnp.float32),
    mesh=_MESH,
    scratch_types=[
        pltpu.VMEM((GS, CH), jnp.int32),
        pltpu.VMEM((GS, CH), jnp.int32),
        pltpu.VMEM((CH, D), jnp.float32),
        pltpu.VMEM((CH, D), jnp.float32),
        pltpu.VMEM_SHARED((N_PAD, D), jnp.float32),
        pltpu.SemaphoreType.DMA,
        pltpu.SemaphoreType.DMA,
    ],
)
def _sc_agg(srcp_hbm, dstp_hbm, y_hbm, out_hbm, src_v, dst_v, buf0, buf1,
            acc_sh, sem0, sem1):
    c = lax.axis_index("c")
    s = lax.axis_index("s")
    r0 = pl.multiple_of(s * RPT, 8)
    pltpu.sync_copy(y_hbm.at[pl.ds(r0, RPT)], acc_sh.at[pl.ds(r0, RPT)])

    @pl.when(s == 0)
    def _seed_rem():
        pltpu.sync_copy(y_hbm.at[pl.ds(RBASE, RREM)],
                        acc_sh.at[pl.ds(RBASE, RREM)])

    plsc.subcore_barrier()

    def run_edges(base, nstage):
        for g in range(nstage):
            off = pl.multiple_of(base + g * GS, 8)
            pltpu.sync_copy(srcp_hbm.at[pl.ds(off, GS)], src_v)
            pltpu.sync_copy(dstp_hbm.at[pl.ds(off, GS)], dst_v)
            pltpu.async_copy(y_hbm.at[src_v.at[0]], buf0, sem0)

            def body(k, carry):
                j0 = 2 * k
                pltpu.async_copy(y_hbm.at[src_v.at[j0 + 1]], buf1, sem1)
                pltpu.make_async_copy(y_hbm.at[src_v.at[j0]], buf0, sem0).wait()
                pltpu.sync_copy(buf0, acc_sh.at[dst_v.at[j0]], add=True)
                pltpu.async_copy(y_hbm.at[src_v.at[(j0 + 2) % GS]], buf0, sem0)
                pltpu.make_async_copy(y_hbm.at[src_v.at[j0 + 1]], buf1,
                                      sem1).wait()
                pltpu.sync_copy(buf1, acc_sh.at[dst_v.at[j0 + 1]], add=True)
                return carry

            lax.fori_loop(0, GS // 2, body, 0)
            pltpu.make_async_copy(y_hbm.at[src_v.at[0]], buf0, sem0).wait()

    run_edges((s * NC + c) * CPT, CPT // GS)
    plsc.subcore_barrier()
    pltpu.sync_copy(acc_sh.at[pl.ds(r0, RPT)],
                    out_hbm.at[c, pl.ds(r0, RPT)])

    @pl.when(s == 0)
    def _out_rem():
        pltpu.sync_copy(acc_sh.at[pl.ds(RBASE, RREM)],
                        out_hbm.at[c, pl.ds(RBASE, RREM)])


_BN = 1000
_GRID = N // _BN


def _tc_first_body(degp_ref, x_ref, w_ref, dis_ref, y_ref):
    deg = degp_ref[0] + degp_ref[1] + 1.0
    dis = lax.rsqrt(deg)
    dis_ref[...] = dis
    y_ref[...] = dis * jnp.dot(x_ref[...], w_ref[...],
                               preferred_element_type=jnp.float32)


def _tc_first(degp, x, w):
    return pl.pallas_call(
        _tc_first_body,
        grid=(_GRID,),
        in_specs=[
            pl.BlockSpec((NC, _BN, 1), lambda i: (0, i, 0)),
            pl.BlockSpec((_BN, D), lambda i: (i, 0)),
            pl.BlockSpec((D, D), lambda i: (0, 0)),
        ],
        out_specs=[
            pl.BlockSpec((_BN, 1), lambda i: (i, 0)),
            pl.BlockSpec((_BN, D), lambda i: (i, 0)),
        ],
        out_shape=[
            jax.ShapeDtypeStruct((N, 1), jnp.float32),
            jax.ShapeDtypeStruct((N, D), jnp.float32),
        ],
    )(degp, x, w)


def _tc_mid_body(z_ref, yin_ref, dis_ref, b_ref, w_ref, y_ref):
    h = jnp.maximum(
        dis_ref[...] * (z_ref[0] + z_ref[1] - yin_ref[...]) + b_ref[...], 0.0)
    y_ref[...] = dis_ref[...] * jnp.dot(h, w_ref[...],
                                        preferred_element_type=jnp.float32)


def _tc_mid(z, yin, dis, b2d, w):
    return pl.pallas_call(
        _tc_mid_body,
        grid=(_GRID,),
        in_specs=[
            pl.BlockSpec((NC, _BN, D), lambda i: (0, i, 0)),
            pl.BlockSpec((_BN, D), lambda i: (i, 0)),
            pl.BlockSpec((_BN, 1), lambda i: (i, 0)),
            pl.BlockSpec((1, D), lambda i: (0, 0)),
            pl.BlockSpec((D, D), lambda i: (0, 0)),
        ],
        out_specs=pl.BlockSpec((_BN, D), lambda i: (i, 0)),
        out_shape=jax.ShapeDtypeStruct((N, D), jnp.float32),
    )(z, yin, dis, b2d, w)


def _tc_final_body(z_ref, yin_ref, dis_ref, b_ref, batch_ref, l1w_ref,
                   l1b_ref, l2w_ref, l2b_ref, out_ref, sums, counts):
    i = pl.program_id(0)

    @pl.when(i == 0)
    def _init():
        sums[...] = jnp.zeros_like(sums)
        counts[...] = jnp.zeros_like(counts)

    h = jnp.maximum(
        dis_ref[...] * (z_ref[0] + z_ref[1] - yin_ref[...]) + b_ref[...], 0.0)
    iota = lax.broadcasted_iota(jnp.int32, (G, 1), 0)
    onehot = (batch_ref[0] == iota).astype(jnp.float32)
    sums[...] += jnp.dot(onehot, h, preferred_element_type=jnp.float32)
    counts[...] += jnp.dot(onehot, jnp.ones((_BN, 1), jnp.float32),
                           preferred_element_type=jnp.float32)

    @pl.when(i == _GRID - 1)
    def _finish():
        pooled = sums[...] / jnp.maximum(counts[...], 1.0)
        a = jnp.maximum(
            jnp.dot(pooled, l1w_ref[...], preferred_element_type=jnp.float32)
            + l1b_ref[...], 0.0)
        out_ref[...] = (jnp.dot(a, l2w_ref[...],
                                preferred_element_type=jnp.float32)
                        + l2b_ref[...])


def _tc_final(z, yin, dis, b2d, batch3, l1w, l1b2d, l2w, l2b2d):
    return pl.pallas_call(
        _tc_final_body,
        grid=(_GRID,),
        in_specs=[
            pl.BlockSpec((NC, _BN, D), lambda i: (0, i, 0)),
            pl.BlockSpec((_BN, D), lambda i: (i, 0)),
            pl.BlockSpec((_BN, 1), lambda i: (i, 0)),
            pl.BlockSpec((1, D), lambda i: (0, 0)),
            pl.BlockSpec((1, 1, _BN), lambda i: (i, 0, 0)),
            pl.BlockSpec((D, HID), lambda i: (0, 0)),
            pl.BlockSpec((1, HID), lambda i: (0, 0)),
            pl.BlockSpec((HID, 1), lambda i: (0, 0)),
            pl.BlockSpec((1, 1), lambda i: (0, 0)),
        ],
        out_specs=pl.BlockSpec((G, 1), lambda i: (0, 0)),
        out_shape=jax.ShapeDtypeStruct((G, 1), jnp.float32),
        scratch_shapes=[
            pltpu.VMEM((G, D), jnp.float32),
            pltpu.VMEM((G, 1), jnp.float32),
        ],
    )(z, yin, dis, b2d, batch3, l1w, l1b2d, l2w, l2b2d)


def kernel(x, edge_index, batch, W1, b1, W2, b2, W3, b3,
           lin1_W, lin1_b, lin2_W, lin2_b):
    src = edge_index[0]
    dst = edge_index[1]
    pad = E_PAD - E
    srcp = jnp.concatenate(
        [src, jnp.zeros((pad,), jnp.int32)]).reshape(TOT, CH)
    pad_dst = jnp.full((pad,), N, jnp.int32)
    dstp = jnp.concatenate([dst, pad_dst]).reshape(TOT, CH)
    dstp3 = dstp.reshape(NW, CPT, CH)
    zeros_deg = jnp.zeros((N_PAD, DW), jnp.float32)
    ones_ch = jnp.ones((CH, DW), jnp.float32)
    batch3 = batch.reshape(_GRID, 1, _BN)

    degp = _sc_deg(dstp3, zeros_deg, ones_ch)
    degp = degp[:, :, :1]
    dis, y = _tc_first(degp, x, W1)
    z = _sc_agg(srcp, dstp, y)
    y = _tc_mid(z, y, dis, b1.reshape(1, D), W2)
    z = _sc_agg(srcp, dstp, y)
    y = _tc_mid(z, y, dis, b2.reshape(1, D), W3)
    z = _sc_agg(srcp, dstp, y)
    return _tc_final(z, y, dis, b3.reshape(1, D), batch3,
                     lin1_W, lin1_b.reshape(1, HID),
                     lin2_W, lin2_b.reshape(1, 1))


# speedup vs baseline: 1.2916x; 1.1486x over previous
import functools

import jax
import jax.numpy as jnp
from jax import lax
from jax.experimental import pallas as pl
from jax.experimental.pallas import tpu as pltpu
from jax.experimental.pallas import tpu_sc as plsc

N = 10000
E = 320000
D = 128
HID = 256
G = 64

NC = 2
NS = 16
NW = NC * NS

CH = 128
CPT = 80
TOT = CPT * NW
GS = 40
EP = CH * CPT
E_PAD = EP * NW
N_PAD = 10240
RPT = 624
RREM = N - NS * RPT
RBASE = NS * RPT
DRT = N_PAD // NS

_MESH = plsc.VectorSubcoreMesh(
    core_axis_name="c", subcore_axis_name="s", num_cores=NC, num_subcores=NS)


DW = 128


@functools.partial(
    pl.kernel,
    out_type=jax.ShapeDtypeStruct((NC, N_PAD, DW), jnp.float32),
    mesh=_MESH,
    scratch_types=[
        pltpu.VMEM((CPT, CH), jnp.int32),
        pltpu.VMEM((CH, DW), jnp.float32),
        pltpu.VMEM_SHARED((N_PAD, DW), jnp.float32),
    ],
)
def _sc_deg(dstp_hbm, zeros_hbm, ones_hbm, out_hbm, dst_v, ones_v, acc_sh):
    c = lax.axis_index("c")
    s = lax.axis_index("s")
    wid = s * NC + c
    d0 = pl.multiple_of(s * DRT, 8)
    pltpu.sync_copy(zeros_hbm.at[pl.ds(d0, DRT)],
                    acc_sh.at[pl.ds(d0, DRT)])
    pltpu.sync_copy(ones_hbm, ones_v)
    pltpu.sync_copy(dstp_hbm.at[wid], dst_v)
    plsc.subcore_barrier()

    def body(j, carry):
        pltpu.sync_copy(ones_v, acc_sh.at[dst_v.at[j]], add=True)
        return carry

    lax.fori_loop(0, CPT, body, 0)
    plsc.subcore_barrier()
    pltpu.sync_copy(acc_sh.at[pl.ds(d0, DRT)],
                    out_hbm.at[c, pl.ds(d0, DRT)])


@functools.partial(
    pl.kernel,
    out_type=jax.ShapeDtypeStruct((NC, N, D), jnp.float32),
    mesh=_MESH,
    scratch_types=[
        pltpu.VMEM((GS, CH), jnp.int32),
        pltpu.VMEM((GS, CH), jnp.int32),
        pltpu.VMEM((CH, D), jnp.float32),
        pltpu.VMEM((CH, D), jnp.float32),
        pltpu.VMEM_SHARED((N_PAD, D), jnp.float32),
        pltpu.SemaphoreType.DMA,
        pltpu.SemaphoreType.DMA,
    ],
)
def _sc_agg(srcp_hbm, dstp_hbm, y_hbm, out_hbm, src_v, dst_v, buf0, buf1,
            acc_sh, sem0, sem1):
    c = lax.axis_index("c")
    s = lax.axis_index("s")
    r0 = pl.multiple_of(s * RPT, 8)
    pltpu.sync_copy(y_hbm.at[pl.ds(r0, RPT)], acc_sh.at[pl.ds(r0, RPT)])

    @pl.when(s == 0)
    def _seed_rem():
        pltpu.sync_copy(y_hbm.at[pl.ds(RBASE, RREM)],
                        acc_sh.at[pl.ds(RBASE, RREM)])

    wid = s * NC + c
    plsc.subcore_barrier()

    for half in range(2):
        pltpu.sync_copy(srcp_hbm.at[wid, pl.ds(half * GS, GS)], src_v)
        pltpu.sync_copy(dstp_hbm.at[wid, pl.ds(half * GS, GS)], dst_v)
        pltpu.async_copy(y_hbm.at[src_v.at[0]], buf0, sem0)

        def body(k, carry):
            j0 = 2 * k
            pltpu.async_copy(y_hbm.at[src_v.at[j0 + 1]], buf1, sem1)
            pltpu.make_async_copy(y_hbm.at[src_v.at[j0]], buf0, sem0).wait()
            pltpu.sync_copy(buf0, acc_sh.at[dst_v.at[j0]], add=True)
            pltpu.async_copy(y_hbm.at[src_v.at[(j0 + 2) % GS]], buf0, sem0)
            pltpu.make_async_copy(y_hbm.at[src_v.at[j0 + 1]], buf1,
                                  sem1).wait()
            pltpu.sync_copy(buf1, acc_sh.at[dst_v.at[j0 + 1]], add=True)
            return carry

        lax.fori_loop(0, GS // 2, body, 0)
        pltpu.make_async_copy(y_hbm.at[src_v.at[0]], buf0, sem0).wait()
    plsc.subcore_barrier()
    pltpu.sync_copy(acc_sh.at[pl.ds(r0, RPT)],
                    out_hbm.at[c, pl.ds(r0, RPT)])

    @pl.when(s == 0)
    def _out_rem():
        pltpu.sync_copy(acc_sh.at[pl.ds(RBASE, RREM)],
                        out_hbm.at[c, pl.ds(RBASE, RREM)])


_BN = 1000
_GRID = N // _BN


def _tc_first_body(degp_ref, x_ref, w_ref, dis_ref, y_ref):
    deg = degp_ref[0] + degp_ref[1] + 1.0
    dis = lax.rsqrt(deg)
    dis_ref[...] = dis
    y_ref[...] = dis * jnp.dot(x_ref[...], w_ref[...],
                               preferred_element_type=jnp.float32)


def _tc_first(degp, x, w):
    return pl.pallas_call(
        _tc_first_body,
        grid=(_GRID,),
        in_specs=[
            pl.BlockSpec((NC, _BN, 1), lambda i: (0, i, 0)),
            pl.BlockSpec((_BN, D), lambda i: (i, 0)),
            pl.BlockSpec((D, D), lambda i: (0, 0)),
        ],
        out_specs=[
            pl.BlockSpec((_BN, 1), lambda i: (i, 0)),
            pl.BlockSpec((_BN, D), lambda i: (i, 0)),
        ],
        out_shape=[
            jax.ShapeDtypeStruct((N, 1), jnp.float32),
            jax.ShapeDtypeStruct((N, D), jnp.float32),
        ],
    )(degp, x, w)


def _tc_mid_body(z_ref, yin_ref, dis_ref, b_ref, w_ref, y_ref):
    h = jnp.maximum(
        dis_ref[...] * (z_ref[0] + z_ref[1] - yin_ref[...]) + b_ref[...], 0.0)
    y_ref[...] = dis_ref[...] * jnp.dot(h, w_ref[...],
                                        preferred_element_type=jnp.float32)


def _tc_mid(z, yin, dis, b2d, w):
    return pl.pallas_call(
        _tc_mid_body,
        grid=(_GRID,),
        in_specs=[
            pl.BlockSpec((NC, _BN, D), lambda i: (0, i, 0)),
            pl.BlockSpec((_BN, D), lambda i: (i, 0)),
            pl.BlockSpec((_BN, 1), lambda i: (i, 0)),
            pl.BlockSpec((1, D), lambda i: (0, 0)),
            pl.BlockSpec((D, D), lambda i: (0, 0)),
        ],
        out_specs=pl.BlockSpec((_BN, D), lambda i: (i, 0)),
        out_shape=jax.ShapeDtypeStruct((N, D), jnp.float32),
    )(z, yin, dis, b2d, w)


def _tc_final_body(z_ref, yin_ref, dis_ref, b_ref, batch_ref, l1w_ref,
                   l1b_ref, l2w_ref, l2b_ref, out_ref, sums, counts):
    i = pl.program_id(0)

    @pl.when(i == 0)
    def _init():
        sums[...] = jnp.zeros_like(sums)
        counts[...] = jnp.zeros_like(counts)

    h = jnp.maximum(
        dis_ref[...] * (z_ref[0] + z_ref[1] - yin_ref[...]) + b_ref[...], 0.0)
    iota = lax.broadcasted_iota(jnp.int32, (G, 1), 0)
    onehot = (batch_ref[0] == iota).astype(jnp.float32)
    sums[...] += jnp.dot(onehot, h, preferred_element_type=jnp.float32)
    counts[...] += jnp.dot(onehot, jnp.ones((_BN, 1), jnp.float32),
                           preferred_element_type=jnp.float32)

    @pl.when(i == _GRID - 1)
    def _finish():
        pooled = sums[...] / jnp.maximum(counts[...], 1.0)
        a = jnp.maximum(
            jnp.dot(pooled, l1w_ref[...], preferred_element_type=jnp.float32)
            + l1b_ref[...], 0.0)
        out_ref[...] = (jnp.dot(a, l2w_ref[...],
                                preferred_element_type=jnp.float32)
                        + l2b_ref[...])


def _tc_final(z, yin, dis, b2d, batch3, l1w, l1b2d, l2w, l2b2d):
    return pl.pallas_call(
        _tc_final_body,
        grid=(_GRID,),
        in_specs=[
            pl.BlockSpec((NC, _BN, D), lambda i: (0, i, 0)),
            pl.BlockSpec((_BN, D), lambda i: (i, 0)),
            pl.BlockSpec((_BN, 1), lambda i: (i, 0)),
            pl.BlockSpec((1, D), lambda i: (0, 0)),
            pl.BlockSpec((1, 1, _BN), lambda i: (i, 0, 0)),
            pl.BlockSpec((D, HID), lambda i: (0, 0)),
            pl.BlockSpec((1, HID), lambda i: (0, 0)),
            pl.BlockSpec((HID, 1), lambda i: (0, 0)),
            pl.BlockSpec((1, 1), lambda i: (0, 0)),
        ],
        out_specs=pl.BlockSpec((G, 1), lambda i: (0, 0)),
        out_shape=jax.ShapeDtypeStruct((G, 1), jnp.float32),
        scratch_shapes=[
            pltpu.VMEM((G, D), jnp.float32),
            pltpu.VMEM((G, 1), jnp.float32),
        ],
    )(z, yin, dis, b2d, batch3, l1w, l1b2d, l2w, l2b2d)


def kernel(x, edge_index, batch, W1, b1, W2, b2, W3, b3,
           lin1_W, lin1_b, lin2_W, lin2_b):
    src = edge_index[0]
    dst = edge_index[1]
    pad = E_PAD - E
    srcp = jnp.concatenate(
        [src, jnp.zeros((pad,), jnp.int32)]).reshape(NW, CPT, CH)
    pad_dst = jnp.full((pad,), N, jnp.int32)
    dstp = jnp.concatenate([dst, pad_dst]).reshape(NW, CPT, CH)
    zeros_deg = jnp.zeros((N_PAD, DW), jnp.float32)
    ones_ch = jnp.ones((CH, DW), jnp.float32)
    batch3 = batch.reshape(_GRID, 1, _BN)

    degp = _sc_deg(dstp, zeros_deg, ones_ch)
    degp = degp[:, :, :1]
    dis, y = _tc_first(degp, x, W1)
    z = _sc_agg(srcp, dstp, y)
    y = _tc_mid(z, y, dis, b1.reshape(1, D), W2)
    z = _sc_agg(srcp, dstp, y)
    y = _tc_mid(z, y, dis, b2.reshape(1, D), W3)
    z = _sc_agg(srcp, dstp, y)
    return _tc_final(z, y, dis, b3.reshape(1, D), batch3,
                     lin1_W, lin1_b.reshape(1, HID),
                     lin2_W, lin2_b.reshape(1, 1))
